# baseline (speedup 1.0000x reference)
import functools

import jax
import jax.numpy as jnp
from jax import lax
from jax.experimental import pallas as pl
from jax.experimental.pallas import tpu as pltpu
from jax.experimental.pallas import tpu_sc as plsc

VOCAB = 1000000
EMBDIM = 32
B = 4096
L = 200

_LGROUPS = 4
_BGROUPS = 8
_LL = L // _LGROUPS
_WB = B // _BGROUPS
_NBT = _WB // 128
_DT = EMBDIM // 8


@functools.partial(
    pl.kernel,
    out_type=jax.ShapeDtypeStruct((L, _DT, B // 128, 8, 128), jnp.float32),
    mesh=plsc.VectorSubcoreMesh(core_axis_name="c", subcore_axis_name="s"),
    compiler_params=pltpu.CompilerParams(
        use_tc_tiling_on_sc=False, needs_layout_passes=False
    ),
    scratch_types=[
        pltpu.VMEM((_LL, _WB), jnp.int32),
        pltpu.VMEM((2, _WB, EMBDIM), jnp.float32),
        pltpu.VMEM((2, _DT, _NBT, 8, 128), jnp.float32),
        pltpu.SemaphoreType.DMA((2,)),
        pltpu.SemaphoreType.DMA((2,)),
    ],
)
def _emb_gather(idx_hbm, table_hbm, out_hbm, idx_v, rows_v, tr_v, gsem, osem):
  wid = lax.axis_index("s") * 2 + lax.axis_index("c")
  lg = wid // jnp.int32(_BGROUPS)
  bg = wid % jnp.int32(_BGROUPS)
  l0 = lg * jnp.int32(_LL)
  b0 = pl.multiple_of(bg * jnp.int32(_WB), _WB)
  bt0 = bg * jnp.int32(_NBT)
  lanes = lax.iota(jnp.int32, 16)

  pltpu.sync_copy(idx_hbm.at[pl.ds(l0, _LL), pl.ds(b0, _WB)], idx_v)

  def gather_start(j, b):
    pltpu.async_copy(
        table_hbm.at[idx_v.at[j]],
        rows_v.at[b],
        gsem.at[b],
    )

  def gather_wait(b):
    pltpu.make_async_copy(
        table_hbm.at[idx_v.at[jnp.int32(0)]],
        rows_v.at[b],
        gsem.at[b],
    ).wait()

  def out_start(j, b):
    pltpu.async_copy(
        tr_v.at[b],
        out_hbm.at[l0 + j, pl.ds(0, _DT), pl.ds(bt0, _NBT)],
        osem.at[b],
    )

  def out_wait(b):
    pltpu.make_async_copy(
        tr_v.at[b],
        out_hbm.at[jnp.int32(0), pl.ds(0, _DT), pl.ds(jnp.int32(0), _NBT)],
        osem.at[b],
    ).wait()

  bidx = [
      lanes + jnp.int32(bt * 128 + lgr * 16)
      for bt in range(_NBT)
      for lgr in range(8)
  ]

  def transpose_chunk(b):
    rows = rows_v.at[b]

    def per_d(_, d):
      dt = d // jnp.int32(8)
      sub = d % jnp.int32(8)
      col = jnp.full((16,), 0, jnp.int32) + d
      for bt in range(_NBT):
        for lgr in range(8):
          vals = plsc.load_gather(rows, [bidx[bt * 8 + lgr], col])
          tr_v[b, dt, jnp.int32(bt), sub, pl.ds(lgr * 16, 16)] = vals
      return d + jnp.int32(1)

    lax.fori_loop(0, EMBDIM, per_d, jnp.int32(0), unroll=False)

  gather_start(jnp.int32(0), jnp.int32(0))

  def chunk(_, j):
    b = j & jnp.int32(1)
    nb = jnp.int32(1) - b

    @pl.when(j < jnp.int32(_LL - 1))
    def _():
      gather_start(j + jnp.int32(1), nb)

    gather_wait(b)

    @pl.when(j >= jnp.int32(2))
    def _():
      out_wait(b)

    transpose_chunk(b)
    out_start(j, b)
    return j + jnp.int32(1)

  lax.fori_loop(0, _LL, chunk, jnp.int32(0), unroll=False)
  out_wait(jnp.int32(0))
  out_wait(jnp.int32(1))


def kernel(indexseq, table):
  idxt = jnp.asarray(indexseq, jnp.int32).T
  out5d = _emb_gather(idxt, table)
  return out5d.transpose(0, 1, 3, 2, 4).reshape(L, EMBDIM, B).transpose(0, 2, 1)
